# baseline (speedup 1.0000x reference)
import functools

import jax
import jax.numpy as jnp
import numpy as np
from jax import lax
from jax.experimental import pallas as pl
from jax.experimental.pallas import tpu as pltpu
from jax.experimental.pallas import tpu_sc as plsc

N_NODES = 10000
D = 128
N_PAD = 10240
TRASH = N_NODES

NC, NS = 2, 16
NW = NC * NS
CHUNK = 128

_PERM = np.array(
    [q * 32 + (2 * r if r < 16 else 2 * (r - 16) + 1)
     for p in range(D)
     for q, r in [divmod(p, 32)]],
    dtype=np.int32,
)


def _sc_segment_sum(h16, src, dst, n_chunks, zeros_rows, zeros_cnt):
    rows_per_sub = N_PAD // NS
    epw = n_chunks * CHUNK

    mesh = plsc.VectorSubcoreMesh(core_axis_name="c", subcore_axis_name="s")

    @functools.partial(
        pl.kernel,
        out_type=(
            jax.ShapeDtypeStruct((NC, N_PAD, D), jnp.float32),
            jax.ShapeDtypeStruct((NC, N_PAD), jnp.float32),
        ),
        mesh=mesh,
        compiler_params=pltpu.CompilerParams(use_tc_tiling_on_sc=False,
                                             needs_layout_passes=False),
        scratch_types=[
            pltpu.VMEM_SHARED((N_PAD, D), jnp.float32),
            pltpu.VMEM_SHARED((N_PAD,), jnp.float32),
            pltpu.VMEM((CHUNK,), jnp.int32),
            pltpu.VMEM((CHUNK,), jnp.int32),
            pltpu.VMEM((CHUNK,), jnp.int32),
            pltpu.VMEM((CHUNK,), jnp.int32),
            pltpu.VMEM((CHUNK, D // 2), jnp.int32),
            pltpu.VMEM((CHUNK, D // 2), jnp.int32),
            pltpu.VMEM((CHUNK, D), jnp.float32),
            pltpu.VMEM((CHUNK,), jnp.float32),
            pltpu.VMEM((rows_per_sub,), jnp.float32),
            pltpu.SemaphoreType.DMA,
            pltpu.SemaphoreType.DMA,
            pltpu.SemaphoreType.DMA,
            pltpu.SemaphoreType.DMA,
        ],
    )
    def seg_kernel(h_hbm, src_hbm, dst_hbm, zr_hbm, zc_hbm,
                   s_out, c_out, s_sh, c_sh, sv0, sv1, dv0, dv1, rb0, rb1,
                   rf32, ones_v, cwb, gsem0, gsem1, isem0, isem1):
        c = lax.axis_index("c")
        s = lax.axis_index("s")
        wid = c * NS + s
        row0 = s * rows_per_sub
        base0 = wid * epw

        pltpu.sync_copy(zr_hbm, rf32)
        for j in range(rows_per_sub // CHUNK):
            pltpu.sync_copy(rf32, s_sh.at[pl.ds(row0 + j * CHUNK, CHUNK)])
        pltpu.sync_copy(zc_hbm, cwb)
        pltpu.sync_copy(cwb, c_sh.at[pl.ds(row0, rows_per_sub)])
        for j in range(CHUNK // 16):
            ones_v[pl.ds(j * 16, 16)] = jnp.ones((16,), jnp.float32)
        plsc.subcore_barrier()

        def widen(rb):
            for i in range(CHUNK):
                for q in range(D // 32):
                    vi = rb[i, pl.ds(q * 16, 16)]
                    lo = plsc.bitcast(vi << 16, jnp.float32)
                    hi = plsc.bitcast(vi, jnp.float32)
                    rf32[i, pl.ds(q * 32, 16)] = lo
                    rf32[i, pl.ds(q * 32 + 16, 16)] = hi

        def load_idx_async(g, sv, dv, isem):
            base = base0 + g * CHUNK
            pltpu.async_copy(src_hbm.at[pl.ds(base, CHUNK)], sv, isem)
            pltpu.async_copy(dst_hbm.at[pl.ds(base, CHUNK)], dv, isem)

        def wait_idx(sv, dv, isem):
            pltpu.make_async_copy(src_hbm.at[pl.ds(0, CHUNK)], sv,
                                  isem).wait()
            pltpu.make_async_copy(dst_hbm.at[pl.ds(0, CHUNK)], dv,
                                  isem).wait()

        load_idx_async(0, sv0, dv0, isem0)
        wait_idx(sv0, dv0, isem0)
        pltpu.async_copy(h_hbm.at[sv0], rb0, gsem0)
        load_idx_async(1, sv1, dv1, isem1)

        def half(g, sv, dv, rb, gsem, isem, osv, odv, orb, ogsem, oisem):
            pltpu.make_async_copy(h_hbm.at[sv], rb, gsem).wait()
            wait_idx(osv, odv, oisem)
            pltpu.async_copy(h_hbm.at[osv], orb, ogsem)
            widen(rb)
            pltpu.sync_copy(rf32, s_sh.at[dv], add=True)
            pltpu.sync_copy(ones_v, c_sh.at[dv], add=True)
            load_idx_async(jnp.minimum(g + 2, n_chunks - 1), sv, dv, isem)

        def pair(k, carry):
            g0 = 2 * k
            half(g0, sv0, dv0, rb0, gsem0, isem0,
                 sv1, dv1, rb1, gsem1, isem1)
            half(g0 + 1, sv1, dv1, rb1, gsem1, isem1,
                 sv0, dv0, rb0, gsem0, isem0)
            return carry

        lax.fori_loop(0, n_chunks // 2, pair, 0)
        pltpu.make_async_copy(h_hbm.at[sv0], rb0, gsem0).wait()
        wait_idx(sv1, dv1, isem1)
        plsc.subcore_barrier()

        for j in range(rows_per_sub // CHUNK):
            r = row0 + j * CHUNK
            pltpu.sync_copy(s_sh.at[pl.ds(r, CHUNK)], rf32)
            pltpu.sync_copy(rf32, s_out.at[c, pl.ds(r, CHUNK)])
        pltpu.sync_copy(c_sh.at[pl.ds(row0, rows_per_sub)], cwb)
        pltpu.sync_copy(cwb, c_out.at[c, pl.ds(row0, rows_per_sub)])

    return seg_kernel(h16, src, dst, zeros_rows, zeros_cnt)


BM = 512


def _tc_combine(h, s_parts, c_parts, w_self_t, w_neigh_t, w_neigh_t_perm,
                bias):
    def body(h_ref, s_ref, c_ref, wst_ref, wnt_ref, wntp_ref, b_ref, o_ref):
        hs = h_ref[...]
        sp = s_ref[0] + s_ref[1]
        cnt = c_ref[0] + c_ref[1] + 1.0
        self_p = jnp.dot(hs, wst_ref[...], preferred_element_type=jnp.float32)
        hn = jnp.dot(hs, wnt_ref[...], preferred_element_type=jnp.float32)
        sn = jnp.dot(sp, wntp_ref[...], preferred_element_type=jnp.float32)
        o_ref[...] = self_p + (hn + sn) / cnt + b_ref[...]

    return pl.pallas_call(
        body,
        grid=(N_PAD // BM,),
        in_specs=[
            pl.BlockSpec((BM, D), lambda i: (i, 0)),
            pl.BlockSpec((NC, BM, D), lambda i: (0, i, 0)),
            pl.BlockSpec((NC, BM, 1), lambda i: (0, i, 0)),
            pl.BlockSpec((D, D), lambda i: (0, 0)),
            pl.BlockSpec((D, D), lambda i: (0, 0)),
            pl.BlockSpec((D, D), lambda i: (0, 0)),
            pl.BlockSpec((1, D), lambda i: (0, 0)),
        ],
        out_specs=pl.BlockSpec((BM, D), lambda i: (i, 0)),
        out_shape=jax.ShapeDtypeStruct((N_NODES, D), jnp.float32),
    )(h, s_parts, c_parts, w_self_t, w_neigh_t, w_neigh_t_perm, bias)


def kernel(h, edges, W_self, b_self, W_neigh, b_neigh):
    src = edges[0].astype(jnp.int32)
    dst = edges[1].astype(jnp.int32)
    e = src.shape[0]
    unit = NW * CHUNK * 2
    e_pad = ((e + unit - 1) // unit) * unit
    n_chunks = e_pad // (NW * CHUNK)
    src = jnp.concatenate([src, jnp.zeros((e_pad - e,), jnp.int32)])
    dst = jnp.concatenate([dst, jnp.full((e_pad - e,), TRASH, jnp.int32)])
    h16 = h.astype(jnp.bfloat16)
    h32 = jax.lax.bitcast_convert_type(h16.reshape(N_NODES, D // 2, 2),
                                       jnp.int32)
    zeros_rows = jnp.zeros((CHUNK, D), jnp.float32)
    zeros_cnt = jnp.zeros((N_PAD // NS,), jnp.float32)

    s_parts, c_parts = _sc_segment_sum(h32, src, dst, n_chunks, zeros_rows,
                                       zeros_cnt)

    perm = jnp.asarray(_PERM)
    w_neigh_t = W_neigh.T
    bias = (b_self + b_neigh).reshape(1, D)
    return _tc_combine(h, s_parts, c_parts.reshape(NC, N_PAD, 1),
                       W_self.T, w_neigh_t, w_neigh_t[perm], bias)


# speedup vs baseline: 1.0090x; 1.0090x over previous
import functools

import jax
import jax.numpy as jnp
import numpy as np
from jax import lax
from jax.experimental import pallas as pl
from jax.experimental.pallas import tpu as pltpu
from jax.experimental.pallas import tpu_sc as plsc

N_NODES = 10000
D = 128
N_PAD = 10240
TRASH = N_NODES

NC, NS = 2, 16
NW = NC * NS
CHUNK = 128

_PERM = np.array(
    [q * 32 + (2 * r if r < 16 else 2 * (r - 16) + 1)
     for p in range(D)
     for q, r in [divmod(p, 32)]],
    dtype=np.int32,
)


def _sc_segment_sum(h16, src, dst, n_chunks, zeros_rows, zeros_cnt):
    rows_per_sub = N_PAD // NS
    epw = n_chunks * CHUNK

    mesh = plsc.VectorSubcoreMesh(core_axis_name="c", subcore_axis_name="s")

    @functools.partial(
        pl.kernel,
        out_type=(
            jax.ShapeDtypeStruct((NC, N_PAD, D), jnp.float32),
            jax.ShapeDtypeStruct((NC, N_PAD), jnp.float32),
        ),
        mesh=mesh,
        compiler_params=pltpu.CompilerParams(use_tc_tiling_on_sc=False,
                                             needs_layout_passes=False),
        scratch_types=[
            pltpu.VMEM_SHARED((N_PAD, D), jnp.float32),
            pltpu.VMEM_SHARED((N_PAD,), jnp.float32),
            pltpu.VMEM((CHUNK,), jnp.int32),
            pltpu.VMEM((CHUNK,), jnp.int32),
            pltpu.VMEM((CHUNK,), jnp.int32),
            pltpu.VMEM((CHUNK,), jnp.int32),
            pltpu.VMEM((CHUNK, D // 2), jnp.int32),
            pltpu.VMEM((CHUNK, D // 2), jnp.int32),
            pltpu.VMEM((CHUNK, D), jnp.float32),
            pltpu.VMEM((CHUNK,), jnp.float32),
            pltpu.VMEM((rows_per_sub,), jnp.float32),
            pltpu.SemaphoreType.DMA,
            pltpu.SemaphoreType.DMA,
        ],
    )
    def seg_kernel(h_hbm, src_hbm, dst_hbm, zr_hbm, zc_hbm,
                   s_out, c_out, s_sh, c_sh, sv0, sv1, dv0, dv1, rb0, rb1,
                   rf32, ones_v, cwb, gsem0, gsem1):
        c = lax.axis_index("c")
        s = lax.axis_index("s")
        wid = c * NS + s
        row0 = s * rows_per_sub
        base0 = wid * epw

        pltpu.sync_copy(zr_hbm, rf32)
        for j in range(rows_per_sub // CHUNK):
            pltpu.sync_copy(rf32, s_sh.at[pl.ds(row0 + j * CHUNK, CHUNK)])
        pltpu.sync_copy(zc_hbm, cwb)
        pltpu.sync_copy(cwb, c_sh.at[pl.ds(row0, rows_per_sub)])
        for j in range(CHUNK // 16):
            ones_v[pl.ds(j * 16, 16)] = jnp.ones((16,), jnp.float32)
        plsc.subcore_barrier()

        def widen(rb):
            for i in range(CHUNK):
                for q in range(D // 32):
                    vi = rb[i, pl.ds(q * 16, 16)]
                    lo = plsc.bitcast(vi << 16, jnp.float32)
                    hi = plsc.bitcast(vi, jnp.float32)
                    rf32[i, pl.ds(q * 32, 16)] = lo
                    rf32[i, pl.ds(q * 32 + 16, 16)] = hi

        def load_idx(g, sv, dv):
            base = base0 + g * CHUNK
            pltpu.sync_copy(src_hbm.at[pl.ds(base, CHUNK)], sv)
            pltpu.sync_copy(dst_hbm.at[pl.ds(base, CHUNK)], dv)

        load_idx(0, sv0, dv0)
        pltpu.async_copy(h_hbm.at[sv0], rb0, gsem0)
        load_idx(1, sv1, dv1)

        def half(g, sv, dv, rb, gsem, osv, orb, ogsem):
            pltpu.async_copy(h_hbm.at[osv], orb, ogsem)
            pltpu.make_async_copy(h_hbm.at[sv], rb, gsem).wait()
            widen(rb)
            pltpu.sync_copy(rf32, s_sh.at[dv], add=True)
            pltpu.sync_copy(ones_v, c_sh.at[dv], add=True)
            load_idx(jnp.minimum(g + 2, n_chunks - 1), sv, dv)

        def pair(k, carry):
            g0 = 2 * k
            half(g0, sv0, dv0, rb0, gsem0, sv1, rb1, gsem1)
            half(g0 + 1, sv1, dv1, rb1, gsem1, sv0, rb0, gsem0)
            return carry

        lax.fori_loop(0, n_chunks // 2, pair, 0)
        pltpu.make_async_copy(h_hbm.at[sv0], rb0, gsem0).wait()
        plsc.subcore_barrier()

        for j in range(rows_per_sub // CHUNK):
            r = row0 + j * CHUNK
            pltpu.sync_copy(s_sh.at[pl.ds(r, CHUNK)], rf32)
            pltpu.sync_copy(rf32, s_out.at[c, pl.ds(r, CHUNK)])
        pltpu.sync_copy(c_sh.at[pl.ds(row0, rows_per_sub)], cwb)
        pltpu.sync_copy(cwb, c_out.at[c, pl.ds(row0, rows_per_sub)])

    return seg_kernel(h16, src, dst, zeros_rows, zeros_cnt)


BM = 512


def _tc_combine(h, s_parts, c_parts, w_self_t, w_neigh_t, w_neigh_t_perm,
                bias):
    def body(h_ref, s_ref, c_ref, wst_ref, wnt_ref, wntp_ref, b_ref, o_ref):
        hs = h_ref[...]
        sp = s_ref[0] + s_ref[1]
        cnt = c_ref[0] + c_ref[1] + 1.0
        self_p = jnp.dot(hs, wst_ref[...], preferred_element_type=jnp.float32)
        hn = jnp.dot(hs, wnt_ref[...], preferred_element_type=jnp.float32)
        sn = jnp.dot(sp, wntp_ref[...], preferred_element_type=jnp.float32)
        o_ref[...] = self_p + (hn + sn) / cnt + b_ref[...]

    return pl.pallas_call(
        body,
        grid=(N_PAD // BM,),
        in_specs=[
            pl.BlockSpec((BM, D), lambda i: (i, 0)),
            pl.BlockSpec((NC, BM, D), lambda i: (0, i, 0)),
            pl.BlockSpec((NC, BM, 1), lambda i: (0, i, 0)),
            pl.BlockSpec((D, D), lambda i: (0, 0)),
            pl.BlockSpec((D, D), lambda i: (0, 0)),
            pl.BlockSpec((D, D), lambda i: (0, 0)),
            pl.BlockSpec((1, D), lambda i: (0, 0)),
        ],
        out_specs=pl.BlockSpec((BM, D), lambda i: (i, 0)),
        out_shape=jax.ShapeDtypeStruct((N_NODES, D), jnp.float32),
    )(h, s_parts, c_parts, w_self_t, w_neigh_t, w_neigh_t_perm, bias)


def kernel(h, edges, W_self, b_self, W_neigh, b_neigh):
    src = edges[0].astype(jnp.int32)
    dst = edges[1].astype(jnp.int32)
    e = src.shape[0]
    unit = NW * CHUNK * 2
    e_pad = ((e + unit - 1) // unit) * unit
    n_chunks = e_pad // (NW * CHUNK)
    src = jnp.concatenate([src, jnp.zeros((e_pad - e,), jnp.int32)])
    dst = jnp.concatenate([dst, jnp.full((e_pad - e,), TRASH, jnp.int32)])
    h16 = h.astype(jnp.bfloat16)
    h32 = jax.lax.bitcast_convert_type(h16.reshape(N_NODES, D // 2, 2),
                                       jnp.int32)
    zeros_rows = jnp.zeros((CHUNK, D), jnp.float32)
    zeros_cnt = jnp.zeros((N_PAD // NS,), jnp.float32)

    s_parts, c_parts = _sc_segment_sum(h32, src, dst, n_chunks, zeros_rows,
                                       zeros_cnt)

    perm = jnp.asarray(_PERM)
    w_neigh_t = W_neigh.T
    bias = (b_self + b_neigh).reshape(1, D)
    return _tc_combine(h, s_parts, c_parts.reshape(NC, N_PAD, 1),
                       W_self.T, w_neigh_t, w_neigh_t[perm], bias)


# speedup vs baseline: 1.0815x; 1.0718x over previous
import functools

import jax
import jax.numpy as jnp
import numpy as np
from jax import lax
from jax.experimental import pallas as pl
from jax.experimental.pallas import tpu as pltpu
from jax.experimental.pallas import tpu_sc as plsc

N_NODES = 10000
D = 128
N_PAD = 10240
TRASH = N_NODES

NC, NS = 2, 16
NW = NC * NS
CHUNK = 128

_PERM = np.array(
    [q * 32 + (2 * r if r < 16 else 2 * (r - 16) + 1)
     for p in range(D)
     for q, r in [divmod(p, 32)]],
    dtype=np.int32,
)


def _sc_segment_sum(h16, packed, n_chunks, zeros_rows, zeros_cnt):
    rows_per_sub = N_PAD // NS
    epw = n_chunks * CHUNK

    mesh = plsc.VectorSubcoreMesh(core_axis_name="c", subcore_axis_name="s")

    @functools.partial(
        pl.kernel,
        out_type=(
            jax.ShapeDtypeStruct((NC, N_PAD, D), jnp.float32),
            jax.ShapeDtypeStruct((NC, N_PAD), jnp.float32),
        ),
        mesh=mesh,
        compiler_params=pltpu.CompilerParams(use_tc_tiling_on_sc=False,
                                             needs_layout_passes=False),
        scratch_types=[
            pltpu.VMEM_SHARED((N_PAD, D), jnp.float32),
            pltpu.VMEM_SHARED((N_PAD,), jnp.float32),
            pltpu.VMEM((2 * CHUNK,), jnp.int32),
            pltpu.VMEM((2 * CHUNK,), jnp.int32),
            pltpu.VMEM((CHUNK, D // 2), jnp.int32),
            pltpu.VMEM((CHUNK, D // 2), jnp.int32),
            pltpu.VMEM((CHUNK, D), jnp.float32),
            pltpu.VMEM((CHUNK,), jnp.float32),
            pltpu.VMEM((rows_per_sub,), jnp.float32),
            pltpu.SemaphoreType.DMA,
            pltpu.SemaphoreType.DMA,
        ],
    )
    def seg_kernel(h_hbm, pk_hbm, zr_hbm, zc_hbm,
                   s_out, c_out, s_sh, c_sh, iv0, iv1, rb0, rb1,
                   rf32, ones_v, cwb, gsem0, gsem1):
        c = lax.axis_index("c")
        s = lax.axis_index("s")
        wid = c * NS + s
        row0 = s * rows_per_sub
        cid0 = wid * n_chunks

        pltpu.sync_copy(zr_hbm, rf32)
        for j in range(rows_per_sub // CHUNK):
            pltpu.sync_copy(rf32, s_sh.at[pl.ds(row0 + j * CHUNK, CHUNK)])
        pltpu.sync_copy(zc_hbm, cwb)
        pltpu.sync_copy(cwb, c_sh.at[pl.ds(row0, rows_per_sub)])
        for j in range(CHUNK // 16):
            ones_v[pl.ds(j * 16, 16)] = jnp.ones((16,), jnp.float32)
        plsc.subcore_barrier()

        def widen(rb):
            for i in range(CHUNK):
                for q in range(D // 32):
                    vi = rb[i, pl.ds(q * 16, 16)]
                    lo = plsc.bitcast(vi << 16, jnp.float32)
                    hi = plsc.bitcast(vi, jnp.float32)
                    rf32[i, pl.ds(q * 32, 16)] = lo
                    rf32[i, pl.ds(q * 32 + 16, 16)] = hi

        def load_idx(g, iv):
            pltpu.sync_copy(pk_hbm.at[cid0 + g], iv)

        def src_of(iv):
            return iv.at[pl.ds(0, CHUNK)]

        def dst_of(iv):
            return iv.at[pl.ds(CHUNK, CHUNK)]

        load_idx(0, iv0)
        pltpu.async_copy(h_hbm.at[src_of(iv0)], rb0, gsem0)
        load_idx(1, iv1)

        def half(g, iv, rb, gsem, oiv, orb, ogsem):
            pltpu.async_copy(h_hbm.at[src_of(oiv)], orb, ogsem)
            pltpu.sync_copy(ones_v, c_sh.at[dst_of(iv)], add=True)
            pltpu.make_async_copy(h_hbm.at[src_of(iv)], rb, gsem).wait()
            widen(rb)
            pltpu.sync_copy(rf32, s_sh.at[dst_of(iv)], add=True)
            load_idx(jnp.minimum(g + 2, n_chunks - 1), iv)

        def pair(k, carry):
            g0 = 2 * k
            half(g0, iv0, rb0, gsem0, iv1, rb1, gsem1)
            half(g0 + 1, iv1, rb1, gsem1, iv0, rb0, gsem0)
            return carry

        lax.fori_loop(0, n_chunks // 2, pair, 0)
        pltpu.make_async_copy(h_hbm.at[src_of(iv0)], rb0, gsem0).wait()
        plsc.subcore_barrier()

        for j in range(rows_per_sub // CHUNK):
            r = row0 + j * CHUNK
            pltpu.sync_copy(s_sh.at[pl.ds(r, CHUNK)], rf32)
            pltpu.sync_copy(rf32, s_out.at[c, pl.ds(r, CHUNK)])
        pltpu.sync_copy(c_sh.at[pl.ds(row0, rows_per_sub)], cwb)
        pltpu.sync_copy(cwb, c_out.at[c, pl.ds(row0, rows_per_sub)])

    return seg_kernel(h16, packed, zeros_rows, zeros_cnt)


BM = 512


def _tc_combine(h, s_parts, c_parts, w_self_t, w_neigh_t, w_neigh_t_perm,
                bias):
    def body(h_ref, s_ref, c_ref, wst_ref, wnt_ref, wntp_ref, b_ref, o_ref):
        hs = h_ref[...]
        sp = s_ref[0] + s_ref[1]
        cnt = c_ref[0] + c_ref[1] + 1.0
        self_p = jnp.dot(hs, wst_ref[...], preferred_element_type=jnp.float32)
        hn = jnp.dot(hs, wnt_ref[...], preferred_element_type=jnp.float32)
        sn = jnp.dot(sp, wntp_ref[...], preferred_element_type=jnp.float32)
        o_ref[...] = self_p + (hn + sn) / cnt + b_ref[...]

    return pl.pallas_call(
        body,
        grid=(N_PAD // BM,),
        in_specs=[
            pl.BlockSpec((BM, D), lambda i: (i, 0)),
            pl.BlockSpec((NC, BM, D), lambda i: (0, i, 0)),
            pl.BlockSpec((NC, BM, 1), lambda i: (0, i, 0)),
            pl.BlockSpec((D, D), lambda i: (0, 0)),
            pl.BlockSpec((D, D), lambda i: (0, 0)),
            pl.BlockSpec((D, D), lambda i: (0, 0)),
            pl.BlockSpec((1, D), lambda i: (0, 0)),
        ],
        out_specs=pl.BlockSpec((BM, D), lambda i: (i, 0)),
        out_shape=jax.ShapeDtypeStruct((N_NODES, D), jnp.float32),
    )(h, s_parts, c_parts, w_self_t, w_neigh_t, w_neigh_t_perm, bias)


def kernel(h, edges, W_self, b_self, W_neigh, b_neigh):
    src = edges[0].astype(jnp.int32)
    dst = edges[1].astype(jnp.int32)
    e = src.shape[0]
    unit = NW * CHUNK * 2
    e_pad = ((e + unit - 1) // unit) * unit
    n_chunks = e_pad // (NW * CHUNK)
    src = jnp.concatenate([src, jnp.zeros((e_pad - e,), jnp.int32)])
    dst = jnp.concatenate([dst, jnp.full((e_pad - e,), TRASH, jnp.int32)])
    tot_chunks = e_pad // CHUNK
    packed = jnp.concatenate(
        [src.reshape(tot_chunks, CHUNK), dst.reshape(tot_chunks, CHUNK)],
        axis=1,
    )
    h16 = h.astype(jnp.bfloat16)
    h32 = jax.lax.bitcast_convert_type(h16.reshape(N_NODES, D // 2, 2),
                                       jnp.int32)
    zeros_rows = jnp.zeros((CHUNK, D), jnp.float32)
    zeros_cnt = jnp.zeros((N_PAD // NS,), jnp.float32)

    s_parts, c_parts = _sc_segment_sum(h32, packed, n_chunks, zeros_rows,
                                       zeros_cnt)

    perm = jnp.asarray(_PERM)
    w_neigh_t = W_neigh.T
    bias = (b_self + b_neigh).reshape(1, D)
    return _tc_combine(h, s_parts, c_parts.reshape(NC, N_PAD, 1),
                       W_self.T, w_neigh_t, w_neigh_t[perm], bias)
